# baseline (speedup 1.0000x reference)
import jax
import jax.numpy as jnp
from jax import lax
from jax.experimental import pallas as pl
from jax.experimental.pallas import tpu as pltpu
from jax.experimental.pallas import tpu_sc as plsc

N = 10000
NPAD = 10240
E = 320000
DI = 128
DH = 64
NCLS = 16

NC = 2
NS = 16
NW = NC * NS
EW = E // NW
CNK = 128
G = 4
NGRP = 20
NCHUNK = NGRP * G
EWPAD = NCHUNK * CNK
DUMP = 10016
RPT = NPAD // NS

_MESH = plsc.VectorSubcoreMesh(core_axis_name="c", subcore_axis_name="s")


def _hist_body(sp_hbm, dp_hbm, out_hbm, hs_sh, hd_sh, ones_v, zb_v, idx_v,
               sem_s):
    c = lax.axis_index("c")
    s = lax.axis_index("s")
    w = c * NS + s
    for i in range(CNK // 16):
        ones_v[pl.ds(16 * i, 16)] = jnp.ones((16,), jnp.float32)
    for i in range(RPT // 16):
        zb_v[pl.ds(16 * i, 16)] = jnp.zeros((16,), jnp.float32)
    pltpu.sync_copy(zb_v, hs_sh.at[pl.ds(RPT * s, RPT)])
    pltpu.sync_copy(zb_v, hd_sh.at[pl.ds(RPT * s, RPT)])
    pltpu.sync_copy(sp_hbm.at[w], idx_v.at[0])
    pltpu.sync_copy(dp_hbm.at[w], idx_v.at[1])
    plsc.subcore_barrier()

    def step(g, carry):
        descs = []
        for i in range(G):
            descs.append(pltpu.async_copy(
                ones_v, hs_sh.at[idx_v.at[0, g * G + i]], sem_s, add=True))
            descs.append(pltpu.async_copy(
                ones_v, hd_sh.at[idx_v.at[1, g * G + i]], sem_s, add=True))
        for d in descs:
            d.wait()
        return carry

    lax.fori_loop(0, NCHUNK // G, step, 0)
    plsc.subcore_barrier()
    pltpu.sync_copy(hs_sh.at[pl.ds(RPT * s, RPT)],
                    out_hbm.at[c, 0, pl.ds(RPT * s, RPT)])
    pltpu.sync_copy(hd_sh.at[pl.ds(RPT * s, RPT)],
                    out_hbm.at[c, 1, pl.ds(RPT * s, RPT)])


_hist_kernel = pl.kernel(
    _hist_body,
    out_type=jax.ShapeDtypeStruct((NC, 2, NPAD), jnp.float32),
    mesh=_MESH,
    scratch_types=[
        pltpu.VMEM_SHARED((NPAD,), jnp.float32),
        pltpu.VMEM_SHARED((NPAD,), jnp.float32),
        pltpu.VMEM((CNK,), jnp.float32),
        pltpu.VMEM((RPT,), jnp.float32),
        pltpu.VMEM((2, NCHUNK, CNK), jnp.int32),
        pltpu.SemaphoreType.DMA,
    ],
)


def _seg_body(h_hbm, sp_hbm, dp_hbm, z_hbm, out_hbm,
              agg_sh, idx_v, rows_v, sem_g0, sem_g1, sem_s0, sem_s1):
    c = lax.axis_index("c")
    s = lax.axis_index("s")
    w = c * NS + s
    r0 = RPT * s
    pltpu.sync_copy(z_hbm, agg_sh.at[pl.ds(r0, RPT)])
    pltpu.sync_copy(sp_hbm.at[w], idx_v.at[0])
    pltpu.sync_copy(dp_hbm.at[w], idx_v.at[1])
    plsc.subcore_barrier()

    def gathers(k, p, sem):
        return [pltpu.async_copy(h_hbm.at[idx_v.at[0, k * G + i]],
                                 rows_v.at[p, i], sem)
                for i in range(G)]

    def scatters(k, p, sem):
        return [pltpu.async_copy(rows_v.at[p, i],
                                 agg_sh.at[idx_v.at[1, k * G + i]], sem,
                                 add=True)
                for i in range(G)]

    def step(m, carry):
        a = 2 * m
        ga = gathers(a, 0, sem_g0)
        gb = gathers(a + 1, 1, sem_g1)
        for d in ga:
            d.wait()
        sa = scatters(a, 0, sem_s0)
        for d in gb:
            d.wait()
        sb = scatters(a + 1, 1, sem_s1)
        for d in sa + sb:
            d.wait()
        return carry

    lax.fori_loop(0, NGRP // 2, step, 0)
    plsc.subcore_barrier()
    pltpu.sync_copy(agg_sh.at[pl.ds(r0, RPT)], out_hbm.at[c, pl.ds(r0, RPT)])


_seg_kernel = pl.kernel(
    _seg_body,
    out_type=jax.ShapeDtypeStruct((NC, NPAD, DH), jnp.float32),
    mesh=_MESH,
    scratch_types=[
        pltpu.VMEM_SHARED((NPAD, DH), jnp.float32),
        pltpu.VMEM((2, NCHUNK, CNK), jnp.int32),
        pltpu.VMEM((2, G, CNK, DH), jnp.float32),
        pltpu.SemaphoreType.DMA,
        pltpu.SemaphoreType.DMA,
        pltpu.SemaphoreType.DMA,
        pltpu.SemaphoreType.DMA,
    ],
    compiler_params=pltpu.CompilerParams(use_tc_tiling_on_sc=False),
)


def _mm_body(x_ref, w_ref, deg_ref, h_ref):
    norm = lax.rsqrt(deg_ref[0] + deg_ref[1] + 1.0)
    h = jnp.dot(x_ref[...], w_ref[...], preferred_element_type=jnp.float32)
    h_ref[...] = h * norm


def _ep_body(aggp_ref, h_ref, degd_ref, b_ref, wc_ref, bc_ref, out_ref):
    agg = aggp_ref[0] + aggp_ref[1] + h_ref[...]
    norm = lax.rsqrt(degd_ref[0] + degd_ref[1] + 1.0)
    act = jnp.maximum(agg * norm + b_ref[...], 0.0)
    rid = lax.broadcasted_iota(jnp.int32, (NPAD, 1), 0)
    act = jnp.where(rid < N, act, -jnp.inf)
    hg = jnp.max(act, axis=0, keepdims=True)
    logits = lax.dot_general(hg, wc_ref[...],
                             (((1,), (1,)), ((), ()))) + bc_ref[...]
    m = jnp.max(logits, axis=1, keepdims=True)
    lse = jnp.log(jnp.sum(jnp.exp(logits - m), axis=1, keepdims=True)) + m
    out_ref[...] = logits - lse


def kernel(x, edge_index, W, b, Wc, bc):
    src = edge_index[0].reshape(NW, EW)
    dst = edge_index[1].reshape(NW, EW)
    pad = ((0, 0), (0, EWPAD - EW))
    sp = jnp.pad(src, pad, constant_values=DUMP).reshape(NW, NCHUNK, CNK)
    dp = jnp.pad(dst, pad, constant_values=DUMP).reshape(NW, NCHUNK, CNK)

    deg = _hist_kernel(sp, dp)

    x_pad = jnp.pad(x, ((0, NPAD - N), (0, 0)))
    deg_src = deg[:, 0, :].reshape(NC, NPAD, 1)
    h = pl.pallas_call(
        _mm_body,
        out_shape=jax.ShapeDtypeStruct((NPAD, DH), jnp.float32),
    )(x_pad, W, deg_src)

    zrows = jnp.zeros((RPT, DH), jnp.float32)
    aggp = _seg_kernel(h, sp, dp, zrows)

    deg_dst = deg[:, 1, :].reshape(NC, NPAD, 1)
    out = pl.pallas_call(
        _ep_body,
        out_shape=jax.ShapeDtypeStruct((1, NCLS), jnp.float32),
    )(aggp, h, deg_dst, b.reshape(1, DH), Wc, bc.reshape(1, NCLS))
    return out


# speedup vs baseline: 1.6176x; 1.6176x over previous
import jax
import jax.numpy as jnp
from jax import lax
from jax.experimental import pallas as pl
from jax.experimental.pallas import tpu as pltpu
from jax.experimental.pallas import tpu_sc as plsc

N = 10000
NPAD = 10240
E = 320000
DI = 128
DH = 64
NCLS = 16

NC = 2
NS = 16
NW = NC * NS
EW = E // NW
CNK = 128
G = 4
SEGG = 2
NBODY = 20
NCHUNK = NBODY * 2 * SEGG
EWPAD = NCHUNK * CNK
DUMP = 10016
RPT = NPAD // NS

_MESH = plsc.VectorSubcoreMesh(core_axis_name="c", subcore_axis_name="s")


def _hist_body(sp_hbm, dp_hbm, out_hbm, hs_sh, hd_sh, ones_v, zb_v, idx_v,
               sem_s):
    c = lax.axis_index("c")
    s = lax.axis_index("s")
    w = c * NS + s
    for i in range(CNK // 16):
        ones_v[pl.ds(16 * i, 16)] = jnp.ones((16,), jnp.float32)
    for i in range(RPT // 16):
        zb_v[pl.ds(16 * i, 16)] = jnp.zeros((16,), jnp.float32)
    pltpu.sync_copy(zb_v, hs_sh.at[pl.ds(RPT * s, RPT)])
    pltpu.sync_copy(zb_v, hd_sh.at[pl.ds(RPT * s, RPT)])
    pltpu.sync_copy(sp_hbm.at[w], idx_v.at[0])
    pltpu.sync_copy(dp_hbm.at[w], idx_v.at[1])
    plsc.subcore_barrier()

    def step(g, carry):
        descs = []
        for i in range(G):
            descs.append(pltpu.async_copy(
                ones_v, hs_sh.at[idx_v.at[0, g * G + i]], sem_s, add=True))
            descs.append(pltpu.async_copy(
                ones_v, hd_sh.at[idx_v.at[1, g * G + i]], sem_s, add=True))
        for d in descs:
            d.wait()
        return carry

    lax.fori_loop(0, NCHUNK // G, step, 0)
    plsc.subcore_barrier()
    pltpu.sync_copy(hs_sh.at[pl.ds(RPT * s, RPT)],
                    out_hbm.at[c, 0, pl.ds(RPT * s, RPT)])
    pltpu.sync_copy(hd_sh.at[pl.ds(RPT * s, RPT)],
                    out_hbm.at[c, 1, pl.ds(RPT * s, RPT)])


_hist_kernel = pl.kernel(
    _hist_body,
    out_type=jax.ShapeDtypeStruct((NC, 2, NPAD), jnp.float32),
    mesh=_MESH,
    scratch_types=[
        pltpu.VMEM_SHARED((NPAD,), jnp.float32),
        pltpu.VMEM_SHARED((NPAD,), jnp.float32),
        pltpu.VMEM((CNK,), jnp.float32),
        pltpu.VMEM((RPT,), jnp.float32),
        pltpu.VMEM((2, NCHUNK, CNK), jnp.int32),
        pltpu.SemaphoreType.DMA,
    ],
)


def _seg_body(h_hbm, sp_hbm, dp_hbm, z_hbm, out_hbm,
              h_sh, agg_sh, idx_v, rows_v, sem_g0, sem_g1, sem_s0, sem_s1):
    c = lax.axis_index("c")
    s = lax.axis_index("s")
    w = c * NS + s
    r0 = RPT * s
    pltpu.sync_copy(h_hbm.at[pl.ds(r0, RPT)], h_sh.at[pl.ds(r0, RPT)])
    pltpu.sync_copy(z_hbm, agg_sh.at[pl.ds(r0, RPT)])
    plsc.subcore_barrier()

    def gathers(k, p, sem):
        return [pltpu.async_copy(h_sh.at[idx_v.at[0, k * SEGG + i]],
                                 rows_v.at[p, i], sem)
                for i in range(SEGG)]

    def scatters(k, p, sem):
        return [pltpu.async_copy(rows_v.at[p, i],
                                 agg_sh.at[idx_v.at[1, k * SEGG + i]], sem,
                                 add=True)
                for i in range(SEGG)]

    def step(m, carry):
        pltpu.sync_copy(sp_hbm.at[w, m], idx_v.at[0])
        pltpu.sync_copy(dp_hbm.at[w, m], idx_v.at[1])
        ga = gathers(0, 0, sem_g0)
        gb = gathers(1, 1, sem_g1)
        for d in ga:
            d.wait()
        sa = scatters(0, 0, sem_s0)
        for d in gb:
            d.wait()
        sb = scatters(1, 1, sem_s1)
        for d in sa + sb:
            d.wait()
        return carry

    lax.fori_loop(0, NBODY, step, 0)
    plsc.subcore_barrier()
    pltpu.sync_copy(agg_sh.at[pl.ds(r0, RPT)], out_hbm.at[c, pl.ds(r0, RPT)])


_seg_kernel = pl.kernel(
    _seg_body,
    out_type=jax.ShapeDtypeStruct((NC, NPAD, DH), jnp.float32),
    mesh=_MESH,
    scratch_types=[
        pltpu.VMEM_SHARED((NPAD, DH), jnp.float32),
        pltpu.VMEM_SHARED((NPAD, DH), jnp.float32),
        pltpu.VMEM((2, 2 * SEGG, CNK), jnp.int32),
        pltpu.VMEM((2, SEGG, CNK, DH), jnp.float32),
        pltpu.SemaphoreType.DMA,
        pltpu.SemaphoreType.DMA,
        pltpu.SemaphoreType.DMA,
        pltpu.SemaphoreType.DMA,
    ],
    compiler_params=pltpu.CompilerParams(use_tc_tiling_on_sc=False),
)


def _mm_body(x_ref, w_ref, deg_ref, h_ref):
    norm = lax.rsqrt(deg_ref[0] + deg_ref[1] + 1.0)
    h = jnp.dot(x_ref[...], w_ref[...], preferred_element_type=jnp.float32)
    h_ref[...] = h * norm


def _ep_body(aggp_ref, h_ref, degd_ref, b_ref, wc_ref, bc_ref, out_ref):
    agg = aggp_ref[0] + aggp_ref[1] + h_ref[...]
    norm = lax.rsqrt(degd_ref[0] + degd_ref[1] + 1.0)
    act = jnp.maximum(agg * norm + b_ref[...], 0.0)
    rid = lax.broadcasted_iota(jnp.int32, (NPAD, 1), 0)
    act = jnp.where(rid < N, act, -jnp.inf)
    hg = jnp.max(act, axis=0, keepdims=True)
    logits = lax.dot_general(hg, wc_ref[...],
                             (((1,), (1,)), ((), ()))) + bc_ref[...]
    m = jnp.max(logits, axis=1, keepdims=True)
    lse = jnp.log(jnp.sum(jnp.exp(logits - m), axis=1, keepdims=True)) + m
    out_ref[...] = logits - lse


def kernel(x, edge_index, W, b, Wc, bc):
    src = edge_index[0].reshape(NW, EW)
    dst = edge_index[1].reshape(NW, EW)
    pad = ((0, 0), (0, EWPAD - EW))
    sp = jnp.pad(src, pad, constant_values=DUMP).reshape(NW, NCHUNK, CNK)
    dp = jnp.pad(dst, pad, constant_values=DUMP).reshape(NW, NCHUNK, CNK)

    deg = _hist_kernel(sp, dp)

    x_pad = jnp.pad(x, ((0, NPAD - N), (0, 0)))
    deg_src = deg[:, 0, :].reshape(NC, NPAD, 1)
    h = pl.pallas_call(
        _mm_body,
        out_shape=jax.ShapeDtypeStruct((NPAD, DH), jnp.float32),
    )(x_pad, W, deg_src)

    zrows = jnp.zeros((RPT, DH), jnp.float32)
    sp4 = sp.reshape(NW, NBODY, 2 * SEGG, CNK)
    dp4 = dp.reshape(NW, NBODY, 2 * SEGG, CNK)
    aggp = _seg_kernel(h, sp4, dp4, zrows)

    deg_dst = deg[:, 1, :].reshape(NC, NPAD, 1)
    out = pl.pallas_call(
        _ep_body,
        out_shape=jax.ShapeDtypeStruct((1, NCLS), jnp.float32),
    )(aggp, h, deg_dst, b.reshape(1, DH), Wc, bc.reshape(1, NCLS))
    return out


# speedup vs baseline: 1.9266x; 1.1910x over previous
import jax
import jax.numpy as jnp
from jax import lax
from jax.experimental import pallas as pl
from jax.experimental.pallas import tpu as pltpu
from jax.experimental.pallas import tpu_sc as plsc

N = 10000
NPAD = 10240
E = 320000
DI = 128
DH = 64
NCLS = 16

NC = 2
NS = 16
NW = NC * NS
CNK = 128
ROWS2 = E // CNK
CPW = ROWS2 // NW
EXTRA = ROWS2 - NW * CPW
G = 4
SEGG = 2
NBODY = 19
RPT = NPAD // NS

_MESH = plsc.VectorSubcoreMesh(core_axis_name="c", subcore_axis_name="s")


def _hist_body(sp_hbm, dp_hbm, out_hbm, hs_sh, hd_sh, ones_v, zb_v,
               idx_v, idx_x, sem_s):
    c = lax.axis_index("c")
    s = lax.axis_index("s")
    w = c * NS + s
    for i in range(CNK // 16):
        ones_v[pl.ds(16 * i, 16)] = jnp.ones((16,), jnp.float32)
    for i in range(RPT // 16):
        zb_v[pl.ds(16 * i, 16)] = jnp.zeros((16,), jnp.float32)
    pltpu.sync_copy(zb_v, hs_sh.at[pl.ds(RPT * s, RPT)])
    pltpu.sync_copy(zb_v, hd_sh.at[pl.ds(RPT * s, RPT)])
    pltpu.sync_copy(sp_hbm.at[pl.ds(w * CPW, CPW)], idx_v.at[0])
    pltpu.sync_copy(dp_hbm.at[pl.ds(w * CPW, CPW)], idx_v.at[1])

    @pl.when(w < EXTRA)
    def _():
        pltpu.sync_copy(sp_hbm.at[pl.ds(NW * CPW + w, 1)], idx_x.at[0])
        pltpu.sync_copy(dp_hbm.at[pl.ds(NW * CPW + w, 1)], idx_x.at[1])

    plsc.subcore_barrier()

    def block(chunks):
        descs = []
        for j in chunks:
            descs.append(pltpu.async_copy(
                ones_v, hs_sh.at[idx_v.at[0, j]], sem_s, add=True))
            descs.append(pltpu.async_copy(
                ones_v, hd_sh.at[idx_v.at[1, j]], sem_s, add=True))
        for d in descs:
            d.wait()

    def step(g, carry):
        block([g * G + i for i in range(G)])
        return carry

    lax.fori_loop(0, CPW // G, step, 0)
    block([CPW - 2, CPW - 1])

    @pl.when(w < EXTRA)
    def _():
        da = pltpu.async_copy(ones_v, hs_sh.at[idx_x.at[0, 0]], sem_s,
                              add=True)
        db = pltpu.async_copy(ones_v, hd_sh.at[idx_x.at[1, 0]], sem_s,
                              add=True)
        da.wait()
        db.wait()

    plsc.subcore_barrier()
    pltpu.sync_copy(hs_sh.at[pl.ds(RPT * s, RPT)],
                    out_hbm.at[c, 0, pl.ds(RPT * s, RPT)])
    pltpu.sync_copy(hd_sh.at[pl.ds(RPT * s, RPT)],
                    out_hbm.at[c, 1, pl.ds(RPT * s, RPT)])


_hist_kernel = pl.kernel(
    _hist_body,
    out_type=jax.ShapeDtypeStruct((NC, 2, NPAD), jnp.float32),
    mesh=_MESH,
    scratch_types=[
        pltpu.VMEM_SHARED((NPAD,), jnp.float32),
        pltpu.VMEM_SHARED((NPAD,), jnp.float32),
        pltpu.VMEM((CNK,), jnp.float32),
        pltpu.VMEM((RPT,), jnp.float32),
        pltpu.VMEM((2, CPW, CNK), jnp.int32),
        pltpu.VMEM((2, 1, CNK), jnp.int32),
        pltpu.SemaphoreType.DMA,
    ],
    compiler_params=pltpu.CompilerParams(use_tc_tiling_on_sc=False),
)


def _seg_body(h_hbm, sp_hbm, dp_hbm, z_hbm, o0_hbm, o1_hbm,
              h_sh, agg_sh, idx_v, idx_x, rows_v,
              sem_g0, sem_g1, sem_s0, sem_s1):
    c = lax.axis_index("c")
    s = lax.axis_index("s")
    w = c * NS + s
    r0 = RPT * s

    @pl.when(s < NS - 1)
    def _():
        pltpu.sync_copy(h_hbm.at[pl.ds(r0, RPT)], h_sh.at[pl.ds(r0, RPT)])

    @pl.when(s == NS - 1)
    def _():
        pltpu.sync_copy(h_hbm.at[pl.ds(N - (N - (NS - 1) * RPT), N - (NS - 1) * RPT)],
                        h_sh.at[pl.ds((NS - 1) * RPT, N - (NS - 1) * RPT)])

    pltpu.sync_copy(z_hbm, agg_sh.at[pl.ds(r0, RPT)])

    @pl.when(w < EXTRA)
    def _():
        pltpu.sync_copy(sp_hbm.at[pl.ds(NW * CPW + w, 1)], idx_x.at[0])
        pltpu.sync_copy(dp_hbm.at[pl.ds(NW * CPW + w, 1)], idx_x.at[1])

    plsc.subcore_barrier()

    def gathers(ks, p, sem):
        return [pltpu.async_copy(h_sh.at[idx_v.at[0, k]],
                                 rows_v.at[p, i], sem)
                for i, k in enumerate(ks)]

    def scatters(ks, p, sem):
        return [pltpu.async_copy(rows_v.at[p, i],
                                 agg_sh.at[idx_v.at[1, k]], sem, add=True)
                for i, k in enumerate(ks)]

    def step(m, carry):
        base = w * CPW + 2 * SEGG * m
        pltpu.sync_copy(sp_hbm.at[pl.ds(base, 2 * SEGG)], idx_v.at[0])
        pltpu.sync_copy(dp_hbm.at[pl.ds(base, 2 * SEGG)], idx_v.at[1])
        ga = gathers(range(SEGG), 0, sem_g0)
        gb = gathers(range(SEGG, 2 * SEGG), 1, sem_g1)
        for d in ga:
            d.wait()
        sa = scatters(range(SEGG), 0, sem_s0)
        for d in gb:
            d.wait()
        sb = scatters(range(SEGG, 2 * SEGG), 1, sem_s1)
        for d in sa + sb:
            d.wait()
        return carry

    lax.fori_loop(0, NBODY, step, 0)

    base = w * CPW + 2 * SEGG * NBODY
    pltpu.sync_copy(sp_hbm.at[pl.ds(base, 2)], idx_v.at[0, pl.ds(0, 2)])
    pltpu.sync_copy(dp_hbm.at[pl.ds(base, 2)], idx_v.at[1, pl.ds(0, 2)])
    gt = gathers(range(2), 0, sem_g0)
    for d in gt:
        d.wait()
    st = scatters(range(2), 0, sem_s0)
    for d in st:
        d.wait()

    @pl.when(w < EXTRA)
    def _():
        gx = pltpu.async_copy(h_sh.at[idx_x.at[0, 0]], rows_v.at[1, 0],
                              sem_g1)
        gx.wait()
        sx = pltpu.async_copy(rows_v.at[1, 0], agg_sh.at[idx_x.at[1, 0]],
                              sem_s1, add=True)
        sx.wait()

    plsc.subcore_barrier()

    @pl.when(c == 0)
    def _():
        pltpu.sync_copy(agg_sh.at[pl.ds(r0, RPT)], o0_hbm.at[pl.ds(r0, RPT)])

    @pl.when(c == 1)
    def _():
        pltpu.sync_copy(agg_sh.at[pl.ds(r0, RPT)], o1_hbm.at[pl.ds(r0, RPT)])


_seg_kernel = pl.kernel(
    _seg_body,
    out_type=(jax.ShapeDtypeStruct((NPAD, DH), jnp.float32),
              jax.ShapeDtypeStruct((NPAD, DH), jnp.float32)),
    mesh=_MESH,
    scratch_types=[
        pltpu.VMEM_SHARED((NPAD, DH), jnp.float32),
        pltpu.VMEM_SHARED((NPAD, DH), jnp.float32),
        pltpu.VMEM((2, 2 * SEGG, CNK), jnp.int32),
        pltpu.VMEM((2, 1, CNK), jnp.int32),
        pltpu.VMEM((2, SEGG, CNK, DH), jnp.float32),
        pltpu.SemaphoreType.DMA,
        pltpu.SemaphoreType.DMA,
        pltpu.SemaphoreType.DMA,
        pltpu.SemaphoreType.DMA,
    ],
    compiler_params=pltpu.CompilerParams(use_tc_tiling_on_sc=False),
)


def _mm_body(x_ref, w_ref, deg_ref, h_ref):
    d = deg_ref[0, 0] + deg_ref[1, 0] + 1.0
    norm = lax.rsqrt(d)[:N].reshape(N, 1)
    h = jnp.dot(x_ref[...], w_ref[...], preferred_element_type=jnp.float32)
    h_ref[...] = h * norm


def _ep_body(a0_ref, a1_ref, h_ref, deg_ref, b_ref, wc_ref, bc_ref, out_ref):
    agg = a0_ref[pl.ds(0, N), :] + a1_ref[pl.ds(0, N), :] + h_ref[...]
    d = deg_ref[0, 1] + deg_ref[1, 1] + 1.0
    norm = lax.rsqrt(d)[:N].reshape(N, 1)
    act = jnp.maximum(agg * norm + b_ref[...], 0.0)
    hg = jnp.max(act, axis=0, keepdims=True)
    logits = lax.dot_general(hg, wc_ref[...],
                             (((1,), (1,)), ((), ()))) + bc_ref[...]
    m = jnp.max(logits, axis=1, keepdims=True)
    lse = jnp.log(jnp.sum(jnp.exp(logits - m), axis=1, keepdims=True)) + m
    out_ref[...] = logits - lse


def kernel(x, edge_index, W, b, Wc, bc):
    sp2 = edge_index[0].reshape(ROWS2, CNK)
    dp2 = edge_index[1].reshape(ROWS2, CNK)

    deg = _hist_kernel(sp2, dp2)

    h = pl.pallas_call(
        _mm_body,
        out_shape=jax.ShapeDtypeStruct((N, DH), jnp.float32),
    )(x, W, deg)

    zrows = jnp.zeros((RPT, DH), jnp.float32)
    a0, a1 = _seg_kernel(h, sp2, dp2, zrows)

    out = pl.pallas_call(
        _ep_body,
        out_shape=jax.ShapeDtypeStruct((1, NCLS), jnp.float32),
    )(a0, a1, h, deg, b.reshape(1, DH), Wc, bc.reshape(1, NCLS))
    return out


# speedup vs baseline: 2.1613x; 1.1218x over previous
import jax
import jax.numpy as jnp
from jax import lax
from jax.experimental import pallas as pl
from jax.experimental.pallas import tpu as pltpu
from jax.experimental.pallas import tpu_sc as plsc

N = 10000
NPAD = 10240
E = 320000
DI = 128
DH = 64
NCLS = 16

NC = 2
NS = 16
NW = NC * NS
CNK = 128
ROWS2 = E // CNK
CPW = ROWS2 // NW
EXTRA = ROWS2 - NW * CPW
G = 4
SEGG = 2
NBODY = 19
RPT = NPAD // NS

_MESH = plsc.VectorSubcoreMesh(core_axis_name="c", subcore_axis_name="s")


def _hist_body(e_hbm, out_hbm, hs_sh, hd_sh, ones_v, zb_v,
               idx_v, idx_x, sem_s):
    c = lax.axis_index("c")
    s = lax.axis_index("s")
    w = c * NS + s
    for i in range(CNK // 16):
        ones_v[pl.ds(16 * i, 16)] = jnp.ones((16,), jnp.float32)
    for i in range(RPT // 16):
        zb_v[pl.ds(16 * i, 16)] = jnp.zeros((16,), jnp.float32)
    pltpu.sync_copy(zb_v, hs_sh.at[pl.ds(RPT * s, RPT)])
    pltpu.sync_copy(zb_v, hd_sh.at[pl.ds(RPT * s, RPT)])
    pltpu.sync_copy(e_hbm.at[0, pl.ds(w * CPW, CPW)], idx_v.at[0])
    pltpu.sync_copy(e_hbm.at[1, pl.ds(w * CPW, CPW)], idx_v.at[1])

    @pl.when(w < EXTRA)
    def _():
        pltpu.sync_copy(e_hbm.at[0, pl.ds(NW * CPW + w, 1)], idx_x.at[0])
        pltpu.sync_copy(e_hbm.at[1, pl.ds(NW * CPW + w, 1)], idx_x.at[1])

    plsc.subcore_barrier()

    def block(chunks):
        descs = []
        for j in chunks:
            descs.append(pltpu.async_copy(
                ones_v, hs_sh.at[idx_v.at[0, j]], sem_s, add=True))
            descs.append(pltpu.async_copy(
                ones_v, hd_sh.at[idx_v.at[1, j]], sem_s, add=True))
        for d in descs:
            d.wait()

    def step(g, carry):
        block([g * G + i for i in range(G)])
        return carry

    lax.fori_loop(0, CPW // G, step, 0)
    block([CPW - 2, CPW - 1])

    @pl.when(w < EXTRA)
    def _():
        da = pltpu.async_copy(ones_v, hs_sh.at[idx_x.at[0, 0]], sem_s,
                              add=True)
        db = pltpu.async_copy(ones_v, hd_sh.at[idx_x.at[1, 0]], sem_s,
                              add=True)
        da.wait()
        db.wait()

    plsc.subcore_barrier()
    pltpu.sync_copy(hs_sh.at[pl.ds(RPT * s, RPT)],
                    out_hbm.at[c, 0, pl.ds(RPT * s, RPT)])
    pltpu.sync_copy(hd_sh.at[pl.ds(RPT * s, RPT)],
                    out_hbm.at[c, 1, pl.ds(RPT * s, RPT)])


_hist_kernel = pl.kernel(
    _hist_body,
    out_type=jax.ShapeDtypeStruct((NC, 2, NPAD), jnp.float32),
    mesh=_MESH,
    scratch_types=[
        pltpu.VMEM_SHARED((NPAD,), jnp.float32),
        pltpu.VMEM_SHARED((NPAD,), jnp.float32),
        pltpu.VMEM((CNK,), jnp.float32),
        pltpu.VMEM((RPT,), jnp.float32),
        pltpu.VMEM((2, CPW, CNK), jnp.int32),
        pltpu.VMEM((2, 1, CNK), jnp.int32),
        pltpu.SemaphoreType.DMA,
    ],
    compiler_params=pltpu.CompilerParams(use_tc_tiling_on_sc=False),
)


def _seg_body(h_hbm, e_hbm, z_hbm, o0_hbm, o1_hbm,
              h_sh, agg_sh, idx_v, idx_x, rows_v,
              sem_g0, sem_g1, sem_s0, sem_s1, sem_i):
    c = lax.axis_index("c")
    s = lax.axis_index("s")
    w = c * NS + s
    r0 = RPT * s

    @pl.when(s < NS - 1)
    def _():
        pltpu.sync_copy(h_hbm.at[pl.ds(r0, RPT)], h_sh.at[pl.ds(r0, RPT)])

    @pl.when(s == NS - 1)
    def _():
        pltpu.sync_copy(h_hbm.at[pl.ds((NS - 1) * RPT, N - (NS - 1) * RPT)],
                        h_sh.at[pl.ds((NS - 1) * RPT, N - (NS - 1) * RPT)])

    pltpu.sync_copy(z_hbm, agg_sh.at[pl.ds(r0, RPT)])

    @pl.when(w < EXTRA)
    def _():
        pltpu.sync_copy(e_hbm.at[0, pl.ds(NW * CPW + w, 1)], idx_x.at[0])
        pltpu.sync_copy(e_hbm.at[1, pl.ds(NW * CPW + w, 1)], idx_x.at[1])

    def load_idx(m, q):
        base = w * CPW + 2 * SEGG * m
        pltpu.async_copy(e_hbm.at[0, pl.ds(base, 2 * SEGG)],
                         idx_v.at[q, 0], sem_i)
        pltpu.async_copy(e_hbm.at[1, pl.ds(base, 2 * SEGG)],
                         idx_v.at[q, 1], sem_i)

    def wait_idx(q):
        for j in range(2):
            pltpu.make_async_copy(e_hbm.at[0, pl.ds(0, 2 * SEGG)],
                                  idx_v.at[q, j], sem_i).wait()

    load_idx(0, 0)
    plsc.subcore_barrier()

    def gathers(q, ks, p, sem):
        return [pltpu.async_copy(h_sh.at[idx_v.at[q, 0, k]],
                                 rows_v.at[p, i], sem)
                for i, k in enumerate(ks)]

    def scatters(q, ks, p, sem):
        return [pltpu.async_copy(rows_v.at[p, i],
                                 agg_sh.at[idx_v.at[q, 1, k]], sem, add=True)
                for i, k in enumerate(ks)]

    def step(m, carry):
        q = lax.rem(m, 2)
        wait_idx(q)

        @pl.when(m + 1 < NBODY)
        def _():
            load_idx(m + 1, 1 - q)

        ga = gathers(q, range(SEGG), 0, sem_g0)
        gb = gathers(q, range(SEGG, 2 * SEGG), 1, sem_g1)
        for d in ga:
            d.wait()
        sa = scatters(q, range(SEGG), 0, sem_s0)
        for d in gb:
            d.wait()
        sb = scatters(q, range(SEGG, 2 * SEGG), 1, sem_s1)
        for d in sa + sb:
            d.wait()
        return carry

    lax.fori_loop(0, NBODY, step, 0)

    base = w * CPW + 2 * SEGG * NBODY
    pltpu.sync_copy(e_hbm.at[0, pl.ds(base, 2)], idx_v.at[0, 0, pl.ds(0, 2)])
    pltpu.sync_copy(e_hbm.at[1, pl.ds(base, 2)], idx_v.at[0, 1, pl.ds(0, 2)])
    gt = gathers(0, range(2), 0, sem_g0)
    for d in gt:
        d.wait()
    st = scatters(0, range(2), 0, sem_s0)
    for d in st:
        d.wait()

    @pl.when(w < EXTRA)
    def _():
        gx = pltpu.async_copy(h_sh.at[idx_x.at[0, 0]], rows_v.at[1, 0],
                              sem_g1)
        gx.wait()
        sx = pltpu.async_copy(rows_v.at[1, 0], agg_sh.at[idx_x.at[1, 0]],
                              sem_s1, add=True)
        sx.wait()

    plsc.subcore_barrier()

    @pl.when(c == 0)
    def _():
        pltpu.sync_copy(agg_sh.at[pl.ds(r0, RPT)], o0_hbm.at[pl.ds(r0, RPT)])

    @pl.when(c == 1)
    def _():
        pltpu.sync_copy(agg_sh.at[pl.ds(r0, RPT)], o1_hbm.at[pl.ds(r0, RPT)])


_seg_kernel = pl.kernel(
    _seg_body,
    out_type=(jax.ShapeDtypeStruct((NPAD, DH), jnp.float32),
              jax.ShapeDtypeStruct((NPAD, DH), jnp.float32)),
    mesh=_MESH,
    scratch_types=[
        pltpu.VMEM_SHARED((NPAD, DH), jnp.float32),
        pltpu.VMEM_SHARED((NPAD, DH), jnp.float32),
        pltpu.VMEM((2, 2, 2 * SEGG, CNK), jnp.int32),
        pltpu.VMEM((2, 1, CNK), jnp.int32),
        pltpu.VMEM((2, SEGG, CNK, DH), jnp.float32),
        pltpu.SemaphoreType.DMA,
        pltpu.SemaphoreType.DMA,
        pltpu.SemaphoreType.DMA,
        pltpu.SemaphoreType.DMA,
        pltpu.SemaphoreType.DMA,
    ],
    compiler_params=pltpu.CompilerParams(use_tc_tiling_on_sc=False),
)


def _mm_body(x_ref, w_ref, deg_ref, h_ref):
    d = deg_ref[0, 0] + deg_ref[1, 0] + 1.0
    norm = lax.rsqrt(d)[:N].reshape(N, 1)
    h = jnp.dot(x_ref[...], w_ref[...], preferred_element_type=jnp.float32)
    h_ref[...] = h * norm


def _ep_body(a0_ref, a1_ref, h_ref, deg_ref, b_ref, wc_ref, bc_ref, out_ref):
    agg = a0_ref[pl.ds(0, N), :] + a1_ref[pl.ds(0, N), :] + h_ref[...]
    d = deg_ref[0, 1] + deg_ref[1, 1] + 1.0
    norm = lax.rsqrt(d)[:N].reshape(N, 1)
    act = jnp.maximum(agg * norm + b_ref[...], 0.0)
    hg = jnp.max(act, axis=0, keepdims=True)
    logits = lax.dot_general(hg, wc_ref[...],
                             (((1,), (1,)), ((), ()))) + bc_ref[...]
    m = jnp.max(logits, axis=1, keepdims=True)
    lse = jnp.log(jnp.sum(jnp.exp(logits - m), axis=1, keepdims=True)) + m
    out_ref[...] = logits - lse


def kernel(x, edge_index, W, b, Wc, bc):
    e3 = edge_index.reshape(2, ROWS2, CNK)

    deg = _hist_kernel(e3)

    h = pl.pallas_call(
        _mm_body,
        out_shape=jax.ShapeDtypeStruct((N, DH), jnp.float32),
    )(x, W, deg)

    zrows = jnp.zeros((RPT, DH), jnp.float32)
    a0, a1 = _seg_kernel(h, e3, zrows)

    out = pl.pallas_call(
        _ep_body,
        out_shape=jax.ShapeDtypeStruct((1, NCLS), jnp.float32),
    )(a0, a1, h, deg, b.reshape(1, DH), Wc, bc.reshape(1, NCLS))
    return out


# speedup vs baseline: 2.1791x; 1.0083x over previous
import jax
import jax.numpy as jnp
from jax import lax
from jax.experimental import pallas as pl
from jax.experimental.pallas import tpu as pltpu
from jax.experimental.pallas import tpu_sc as plsc

N = 10000
NPAD = 10240
E = 320000
DI = 128
DH = 64
NCLS = 16

NC = 2
NS = 16
NW = NC * NS
CNK = 128
ROWS2 = E // CNK
CPW = ROWS2 // NW
EXTRA = ROWS2 - NW * CPW
G = 6
SEGG = 2
NBODY = 19
RPT = NPAD // NS

_MESH = plsc.VectorSubcoreMesh(core_axis_name="c", subcore_axis_name="s")


def _hist_body(e_hbm, out_hbm, hs_sh, hd_sh, ones_v, zb_v,
               idx_v, idx_x, sem_s):
    c = lax.axis_index("c")
    s = lax.axis_index("s")
    w = c * NS + s
    for i in range(CNK // 16):
        ones_v[pl.ds(16 * i, 16)] = jnp.ones((16,), jnp.float32)
    for i in range(RPT // 16):
        zb_v[pl.ds(16 * i, 16)] = jnp.zeros((16,), jnp.float32)
    pltpu.sync_copy(zb_v, hs_sh.at[pl.ds(RPT * s, RPT)])
    pltpu.sync_copy(zb_v, hd_sh.at[pl.ds(RPT * s, RPT)])
    pltpu.sync_copy(e_hbm.at[0, pl.ds(w * CPW, CPW)], idx_v.at[0])
    pltpu.sync_copy(e_hbm.at[1, pl.ds(w * CPW, CPW)], idx_v.at[1])

    @pl.when(w < EXTRA)
    def _():
        pltpu.sync_copy(e_hbm.at[0, pl.ds(NW * CPW + w, 1)], idx_x.at[0])
        pltpu.sync_copy(e_hbm.at[1, pl.ds(NW * CPW + w, 1)], idx_x.at[1])

    plsc.subcore_barrier()

    def block(chunks):
        descs = []
        for j in chunks:
            descs.append(pltpu.async_copy(
                ones_v, hs_sh.at[idx_v.at[0, j]], sem_s, add=True))
            descs.append(pltpu.async_copy(
                ones_v, hd_sh.at[idx_v.at[1, j]], sem_s, add=True))
        for d in descs:
            d.wait()

    def step(g, carry):
        block([g * G + i for i in range(G)])
        return carry

    lax.fori_loop(0, CPW // G, step, 0)

    @pl.when(w < EXTRA)
    def _():
        da = pltpu.async_copy(ones_v, hs_sh.at[idx_x.at[0, 0]], sem_s,
                              add=True)
        db = pltpu.async_copy(ones_v, hd_sh.at[idx_x.at[1, 0]], sem_s,
                              add=True)
        da.wait()
        db.wait()

    plsc.subcore_barrier()
    pltpu.sync_copy(hs_sh.at[pl.ds(RPT * s, RPT)],
                    out_hbm.at[c, 0, pl.ds(RPT * s, RPT)])
    pltpu.sync_copy(hd_sh.at[pl.ds(RPT * s, RPT)],
                    out_hbm.at[c, 1, pl.ds(RPT * s, RPT)])


_hist_kernel = pl.kernel(
    _hist_body,
    out_type=jax.ShapeDtypeStruct((NC, 2, NPAD), jnp.float32),
    mesh=_MESH,
    scratch_types=[
        pltpu.VMEM_SHARED((NPAD,), jnp.float32),
        pltpu.VMEM_SHARED((NPAD,), jnp.float32),
        pltpu.VMEM((CNK,), jnp.float32),
        pltpu.VMEM((RPT,), jnp.float32),
        pltpu.VMEM((2, CPW, CNK), jnp.int32),
        pltpu.VMEM((2, 1, CNK), jnp.int32),
        pltpu.SemaphoreType.DMA,
    ],
    compiler_params=pltpu.CompilerParams(use_tc_tiling_on_sc=False),
)


def _seg_body(h_hbm, e_hbm, z_hbm, o0_hbm, o1_hbm,
              h_sh, agg_sh, idx_v, idx_x, rows_v,
              sem_g0, sem_g1, sem_s0, sem_s1, sem_i):
    c = lax.axis_index("c")
    s = lax.axis_index("s")
    w = c * NS + s
    r0 = RPT * s

    @pl.when(s < NS - 1)
    def _():
        dh = pltpu.async_copy(h_hbm.at[pl.ds(r0, RPT)],
                              h_sh.at[pl.ds(r0, RPT)], sem_i)
        dz = pltpu.async_copy(z_hbm, agg_sh.at[pl.ds(r0, RPT)], sem_g0)
        dh.wait()
        dz.wait()

    @pl.when(s == NS - 1)
    def _():
        lastn = N - (NS - 1) * RPT
        dh = pltpu.async_copy(h_hbm.at[pl.ds((NS - 1) * RPT, lastn)],
                              h_sh.at[pl.ds((NS - 1) * RPT, lastn)], sem_i)
        dz = pltpu.async_copy(z_hbm, agg_sh.at[pl.ds(r0, RPT)], sem_g0)
        dh.wait()
        dz.wait()

    @pl.when(w < EXTRA)
    def _():
        pltpu.sync_copy(e_hbm.at[0, pl.ds(NW * CPW + w, 1)], idx_x.at[0])
        pltpu.sync_copy(e_hbm.at[1, pl.ds(NW * CPW + w, 1)], idx_x.at[1])

    def load_idx(m, q):
        base = w * CPW + 2 * SEGG * m
        pltpu.async_copy(e_hbm.at[0, pl.ds(base, 2 * SEGG)],
                         idx_v.at[q, 0], sem_i)
        pltpu.async_copy(e_hbm.at[1, pl.ds(base, 2 * SEGG)],
                         idx_v.at[q, 1], sem_i)

    def wait_idx(q):
        for j in range(2):
            pltpu.make_async_copy(e_hbm.at[0, pl.ds(0, 2 * SEGG)],
                                  idx_v.at[q, j], sem_i).wait()

    load_idx(0, 0)
    plsc.subcore_barrier()

    def gathers(q, ks, p, sem):
        return [pltpu.async_copy(h_sh.at[idx_v.at[q, 0, k]],
                                 rows_v.at[p, i], sem)
                for i, k in enumerate(ks)]

    def scatters(q, ks, p, sem):
        return [pltpu.async_copy(rows_v.at[p, i],
                                 agg_sh.at[idx_v.at[q, 1, k]], sem, add=True)
                for i, k in enumerate(ks)]

    def step(m, carry):
        q = lax.rem(m, 2)
        wait_idx(q)

        @pl.when(m + 1 < NBODY)
        def _():
            load_idx(m + 1, 1 - q)

        ga = gathers(q, range(SEGG), 0, sem_g0)
        gb = gathers(q, range(SEGG, 2 * SEGG), 1, sem_g1)
        for d in ga:
            d.wait()
        sa = scatters(q, range(SEGG), 0, sem_s0)
        for d in gb:
            d.wait()
        sb = scatters(q, range(SEGG, 2 * SEGG), 1, sem_s1)
        for d in sa + sb:
            d.wait()
        return carry

    lax.fori_loop(0, NBODY, step, 0)

    base = w * CPW + 2 * SEGG * NBODY
    pltpu.sync_copy(e_hbm.at[0, pl.ds(base, 2)], idx_v.at[0, 0, pl.ds(0, 2)])
    pltpu.sync_copy(e_hbm.at[1, pl.ds(base, 2)], idx_v.at[0, 1, pl.ds(0, 2)])
    gt = gathers(0, range(2), 0, sem_g0)
    for d in gt:
        d.wait()
    st = scatters(0, range(2), 0, sem_s0)
    for d in st:
        d.wait()

    @pl.when(w < EXTRA)
    def _():
        gx = pltpu.async_copy(h_sh.at[idx_x.at[0, 0]], rows_v.at[1, 0],
                              sem_g1)
        gx.wait()
        sx = pltpu.async_copy(rows_v.at[1, 0], agg_sh.at[idx_x.at[1, 0]],
                              sem_s1, add=True)
        sx.wait()

    plsc.subcore_barrier()

    @pl.when(c == 0)
    def _():
        pltpu.sync_copy(agg_sh.at[pl.ds(r0, RPT)], o0_hbm.at[pl.ds(r0, RPT)])

    @pl.when(c == 1)
    def _():
        pltpu.sync_copy(agg_sh.at[pl.ds(r0, RPT)], o1_hbm.at[pl.ds(r0, RPT)])


_seg_kernel = pl.kernel(
    _seg_body,
    out_type=(jax.ShapeDtypeStruct((NPAD, DH), jnp.float32),
              jax.ShapeDtypeStruct((NPAD, DH), jnp.float32)),
    mesh=_MESH,
    scratch_types=[
        pltpu.VMEM_SHARED((NPAD, DH), jnp.float32),
        pltpu.VMEM_SHARED((NPAD, DH), jnp.float32),
        pltpu.VMEM((2, 2, 2 * SEGG, CNK), jnp.int32),
        pltpu.VMEM((2, 1, CNK), jnp.int32),
        pltpu.VMEM((2, SEGG, CNK, DH), jnp.float32),
        pltpu.SemaphoreType.DMA,
        pltpu.SemaphoreType.DMA,
        pltpu.SemaphoreType.DMA,
        pltpu.SemaphoreType.DMA,
        pltpu.SemaphoreType.DMA,
    ],
    compiler_params=pltpu.CompilerParams(use_tc_tiling_on_sc=False),
)


def _mm_body(x_ref, w_ref, deg_ref, h_ref):
    d = deg_ref[0, 0] + deg_ref[1, 0] + 1.0
    norm = lax.rsqrt(d)[:N].reshape(N, 1)
    h = jnp.dot(x_ref[...], w_ref[...], preferred_element_type=jnp.float32)
    h_ref[...] = h * norm


def _ep_body(a0_ref, a1_ref, h_ref, deg_ref, b_ref, wc_ref, bc_ref, out_ref):
    agg = a0_ref[pl.ds(0, N), :] + a1_ref[pl.ds(0, N), :] + h_ref[...]
    d = deg_ref[0, 1] + deg_ref[1, 1] + 1.0
    norm = lax.rsqrt(d)[:N].reshape(N, 1)
    act = jnp.maximum(agg * norm + b_ref[...], 0.0)
    hg = jnp.max(act, axis=0, keepdims=True)
    logits = lax.dot_general(hg, wc_ref[...],
                             (((1,), (1,)), ((), ()))) + bc_ref[...]
    m = jnp.max(logits, axis=1, keepdims=True)
    lse = jnp.log(jnp.sum(jnp.exp(logits - m), axis=1, keepdims=True)) + m
    out_ref[...] = logits - lse


def kernel(x, edge_index, W, b, Wc, bc):
    e3 = edge_index.reshape(2, ROWS2, CNK)

    deg = _hist_kernel(e3)

    h = pl.pallas_call(
        _mm_body,
        out_shape=jax.ShapeDtypeStruct((N, DH), jnp.float32),
    )(x, W, deg)

    zrows = jnp.zeros((RPT, DH), jnp.float32)
    a0, a1 = _seg_kernel(h, e3, zrows)

    out = pl.pallas_call(
        _ep_body,
        out_shape=jax.ShapeDtypeStruct((1, NCLS), jnp.float32),
    )(a0, a1, h, deg, b.reshape(1, DH), Wc, bc.reshape(1, NCLS))
    return out
